# baseline (speedup 1.0000x reference)
import functools

import jax
import jax.numpy as jnp
from jax import lax
from jax.experimental import pallas as pl
from jax.experimental.pallas import tpu as pltpu
from jax.experimental.pallas import tpu_sc as plsc

P = 128
H = 32
E = 1024
NLANE = 16
ROWS_PER_W = 4


def _sc_edge_histogram(src_hbm, dst_hbm, cnt_hbm, src_v, dst_v, acc_v, row_v):
    wid = lax.axis_index("s") * 2 + lax.axis_index("c")
    lo = wid * ROWS_PER_W
    pltpu.sync_copy(src_hbm, src_v)
    pltpu.sync_copy(dst_hbm, dst_v)
    zeros16 = jnp.zeros((NLANE,), jnp.float32)
    ones16 = jnp.ones((NLANE,), jnp.float32)
    lane = lax.broadcasted_iota(jnp.int32, (NLANE,), 0)
    npriv = ROWS_PER_W * P
    for i in range(NLANE * npriv // NLANE):
        acc_v[pl.ds(i * NLANE, NLANE)] = zeros16
    for e in range(0, E, NLANE):
        s16 = src_v[pl.ds(e, NLANE)]
        d16 = dst_v[pl.ds(e, NLANE)]
        m = (d16 >= lo) & (d16 < lo + ROWS_PER_W)
        idx = lane * npriv + (d16 - lo) * P + s16
        idx = jnp.where(m, idx, 0)
        plsc.addupdate_scatter(acc_v, [idx], ones16, mask=m)
    for c in range(npriv // NLANE):
        tot = zeros16
        for l in range(NLANE):
            tot = tot + acc_v[pl.ds(l * npriv + c * NLANE, NLANE)]
        row_v[c // (P // NLANE), pl.ds((c % (P // NLANE)) * NLANE, NLANE)] = tot
    pltpu.sync_copy(row_v, cnt_hbm.at[pl.ds(lo, ROWS_PER_W)])


def _build_cnt(src, dst):
    mesh = plsc.VectorSubcoreMesh(core_axis_name="c", subcore_axis_name="s")
    return pl.kernel(
        _sc_edge_histogram,
        mesh=mesh,
        compiler_params=pltpu.CompilerParams(needs_layout_passes=False),
        out_type=jax.ShapeDtypeStruct((P, P), jnp.float32),
        scratch_types=[
            pltpu.VMEM((E,), jnp.int32),
            pltpu.VMEM((E,), jnp.int32),
            pltpu.VMEM((NLANE * ROWS_PER_W * P,), jnp.float32),
            pltpu.VMEM((ROWS_PER_W, P), jnp.float32),
        ],
    )(src, dst)


def _fused_kernel(cnt_ref, W_enc_ref, b_enc_ref, W_g_ref, b_g_ref,
                  x_ref, out_ref, A_ref, Wc_ref, bias_ref, *, bb):
    @pl.when(pl.program_id(0) == 0)
    def _build_graph():
        cnt = cnt_ref[...]
        out_deg = jnp.sum(cnt, axis=0) + 1.0
        in_deg = jnp.sum(cnt, axis=1) + 1.0
        eye = (jax.lax.broadcasted_iota(jnp.int32, (P, P), 0) ==
               jax.lax.broadcasted_iota(jnp.int32, (P, P), 1)
               ).astype(jnp.float32)
        An = (jax.lax.rsqrt(in_deg)[:, None] * (cnt + eye) *
              jax.lax.rsqrt(out_deg)[None, :])
        A_ref[...] = An
        Wc_ref[...] = jnp.dot(W_enc_ref[...], W_g_ref[...])
        c1 = jnp.dot(b_enc_ref[...], W_g_ref[...])
        bias_ref[...] = jnp.sum(An, axis=1)[:, None] * c1 + b_g_ref[...]

    xb = x_ref[...]
    x3 = xb.T.reshape(P, H, bb)
    u = jax.lax.dot_general(
        A_ref[...], x3, (((1,), (0,)), ((), ())))
    u = jnp.transpose(u, (0, 2, 1))
    w = jnp.dot(u.reshape(P * bb, H), Wc_ref[...])
    w = jnp.transpose(w.reshape(P, bb, H), (1, 0, 2))
    out_ref[...] = (w + bias_ref[...][None, :, :]).reshape(bb * P, H)


def kernel(x, W_enc, b_enc, W_g, b_g, src, dst):
    B = x.shape[0]
    cnt = _build_cnt(src, dst)
    bb = 128
    out = pl.pallas_call(
        functools.partial(_fused_kernel, bb=bb),
        grid=(B // bb,),
        in_specs=[
            pl.BlockSpec((P, P), lambda i: (0, 0)),
            pl.BlockSpec((H, H), lambda i: (0, 0)),
            pl.BlockSpec((1, H), lambda i: (0, 0)),
            pl.BlockSpec((H, H), lambda i: (0, 0)),
            pl.BlockSpec((1, H), lambda i: (0, 0)),
            pl.BlockSpec((bb, P * H), lambda i: (i, 0)),
        ],
        out_specs=pl.BlockSpec((bb * P, H), lambda i: (i, 0)),
        out_shape=jax.ShapeDtypeStruct((B * P, H), jnp.float32),
        scratch_shapes=[
            pltpu.VMEM((P, P), jnp.float32),
            pltpu.VMEM((H, H), jnp.float32),
            pltpu.VMEM((P, H), jnp.float32),
        ],
    )(cnt, W_enc, b_enc.reshape(1, H), W_g, b_g.reshape(1, H), x)
    return out
